# Initial kernel scaffold; baseline (speedup 1.0000x reference)
import jax
import jax.numpy as jnp
from jax.experimental import pallas as pl


def kernel(x, pe_weight):
    raise NotImplementedError("write your pallas kernel here")


# speedup vs baseline: 1.0125x; 1.0125x over previous
import jax
import jax.numpy as jnp
from jax.experimental import pallas as pl

_ROWS_PER_BLOCK = 512


def _bcast_body(w_ref, out_ref):
    out_ref[...] = jnp.broadcast_to(w_ref[...][None, :, :], out_ref.shape)


def kernel(x, pe_weight):
    batch = x.shape[0]
    maxlen, d = pe_weight.shape
    rb = _ROWS_PER_BLOCK
    return pl.pallas_call(
        _bcast_body,
        grid=(maxlen // rb,),
        in_specs=[pl.BlockSpec((rb, d), lambda i: (i, 0))],
        out_specs=pl.BlockSpec((batch, rb, d), lambda i: (0, i, 0)),
        out_shape=jax.ShapeDtypeStruct((batch, maxlen, d), pe_weight.dtype),
    )(pe_weight)


# speedup vs baseline: 1.0386x; 1.0258x over previous
import jax
import jax.numpy as jnp
from jax.experimental import pallas as pl

_ROWS_PER_BLOCK = 1024


def _bcast_body(w_ref, out_ref):
    out_ref[...] = jnp.broadcast_to(w_ref[...][None, :, :], out_ref.shape)


def kernel(x, pe_weight):
    batch = x.shape[0]
    maxlen, d = pe_weight.shape
    rb = _ROWS_PER_BLOCK
    return pl.pallas_call(
        _bcast_body,
        grid=(maxlen // rb,),
        in_specs=[pl.BlockSpec((rb, d), lambda i: (i, 0))],
        out_specs=pl.BlockSpec((batch, rb, d), lambda i: (0, i, 0)),
        out_shape=jax.ShapeDtypeStruct((batch, maxlen, d), pe_weight.dtype),
    )(pe_weight)
